# baseline (speedup 1.0000x reference)
import functools

import jax
import jax.numpy as jnp
from jax import lax
from jax.experimental import pallas as pl
from jax.experimental.pallas import tpu as pltpu

KNN = 32
TAU2 = 4.0
Q = 1024
D = 32
T = 32
EB = 2048
E_RAW = 100000
NB = (E_RAW + EB - 1) // EB
E_PAD = NB * EB
BIG = 3.0e33

_DOT = functools.partial(
    lax.dot_general,
    preferred_element_type=jnp.float32,
    precision=lax.Precision.HIGHEST,
)


def _dist_block(q, e_raw, base):
    valid = (lax.broadcasted_iota(jnp.int32, (EB, 1), 0) + base) < E_RAW
    e = jnp.where(valid, e_raw, 0.0)
    q2 = jnp.sum(q * q, axis=1, keepdims=True)
    e2 = jnp.sum(e * e, axis=1)[None, :]
    e2 = jnp.where(valid.reshape(1, EB), e2, BIG)
    qe = lax.dot_general(q, e, (((1,), (1,)), ((), ())),
                         preferred_element_type=jnp.float32)
    return jnp.maximum(q2 + e2 - 2.0 * qe, 0.0)


DEPTH = 8
NCLS = 512
RDEPTH = 5


def _make_select(depth):
    def body(q_ref, e_ref, cand_ref):
        d2 = _dist_block(q_ref[...], e_ref[...], pl.program_id(0) * EB)
        cols = []
        for i in range(depth):
            m = jnp.min(d2, axis=1, keepdims=True)
            cols.append(m)
            if i + 1 < depth:
                d2 = jnp.where(d2 <= m, BIG, d2)
        cand_ref[...] = jnp.concatenate(cols, axis=1).reshape(1, Q, depth)
    return body


def _merge_body(cand_ref, t_ref):
    c = cand_ref[...]
    for _ in range(KNN - 1):
        m = jnp.min(c, axis=1, keepdims=True)
        c = jnp.where(c <= m, BIG, c)
    t = jnp.min(c, axis=1, keepdims=True)
    t_ref[...] = jnp.minimum(t, TAU2)


def _stream_select_body(q_ref, e_ref, t_ref, flag_ref, r_ref):
    i = pl.program_id(0)

    @pl.when(i == 0)
    def _init():
        r_ref[...] = jnp.full_like(r_ref, BIG)

    d2 = _dist_block(q_ref[...], e_ref[...], i * EB)
    R = [r_ref[:, j * NCLS:(j + 1) * NCLS] for j in range(RDEPTH)]
    for c in range(EB // NCLS):
        x = d2[:, c * NCLS:(c + 1) * NCLS]
        for j in range(RDEPTH):
            lo = jnp.minimum(R[j], x)
            x = jnp.maximum(R[j], x)
            R[j] = lo
    for j in range(RDEPTH):
        r_ref[:, j * NCLS:(j + 1) * NCLS] = R[j]

    @pl.when(i == NB - 1)
    def _merge():
        c = r_ref[...]
        cm = c
        for _ in range(KNN - 1):
            m = jnp.min(cm, axis=1, keepdims=True)
            cm = jnp.where(cm <= m, BIG, cm)
        t = jnp.minimum(jnp.min(cm, axis=1, keepdims=True), TAU2)
        lane = lax.broadcasted_iota(jnp.int32, c.shape, 1)
        deepest = jnp.where(lane >= (RDEPTH - 1) * NCLS, c, BIG)
        tmin = jnp.min(deepest, axis=1, keepdims=True)
        t_ref[...] = t
        flag_ref[...] = jnp.max(jnp.where(tmin < t, 1.0, 0.0), axis=(0, 1),
                                keepdims=True)


def _accum_body(q_ref, e_ref, lev_ref, lcen_ref, t_ref, bev_ref, bcen_ref,
                out_ref, acc):
    i = pl.program_id(0)

    @pl.when(i == 0)
    def _init():
        acc[...] = jnp.zeros_like(acc)

    d2 = _dist_block(q_ref[...], e_ref[...], i * EB)
    w = jnp.where(d2 <= t_ref[...], jnp.exp(-d2), 0.0)
    valid = (lax.broadcasted_iota(jnp.int32, (EB, 1), 0) + i * EB) < E_RAW
    ev = jnp.where(valid, jnp.exp(lev_ref[...]), 0.0)
    tot = ev + jnp.where(valid, jnp.exp(lcen_ref[...]), 0.0)
    evtot = jnp.concatenate([ev, tot], axis=1)
    acc[...] += lax.dot_general(w, evtot, (((1,), (0,)), ((), ())),
                                preferred_element_type=jnp.float32)

    @pl.when(i == NB - 1)
    def _finish():
        ia = lax.broadcasted_iota(jnp.int32, (T, T), 0)
        ib = lax.broadcasted_iota(jnp.int32, (T, T), 1)
        M = (ia >= ib).astype(jnp.float32)
        bev = jnp.exp(bev_ref[...])
        btot = bev + jnp.exp(bcen_ref[...])
        brisk = _DOT(btot, M, (((1,), (0,)), ((), ())))
        numer = acc[:, :T] + bev
        denom = _DOT(acc[:, T:], M, (((1,), (0,)), ((), ()))) + brisk + 1e-12
        out_ref[...] = jnp.clip(numer / denom, 1e-12, 1.0 - 1e-12)


def _select_threshold(inp, emb_pad, depth):
    cand = pl.pallas_call(
        _make_select(depth),
        grid=(NB,),
        in_specs=[
            pl.BlockSpec((Q, D), lambda i: (0, 0)),
            pl.BlockSpec((EB, D), lambda i: (i, 0)),
        ],
        out_specs=pl.BlockSpec((1, Q, depth), lambda i: (i, 0, 0)),
        out_shape=jax.ShapeDtypeStruct((NB, Q, depth), jnp.float32),
    )(inp, emb_pad)
    return cand.transpose(1, 0, 2).reshape(Q, NB * depth)


def _deep_threshold(inp, emb_pad):
    cand = _select_threshold(inp, emb_pad, KNN)
    return pl.pallas_call(
        _merge_body,
        in_specs=[pl.BlockSpec((Q, NB * KNN), lambda: (0, 0))],
        out_specs=pl.BlockSpec((Q, 1), lambda: (0, 0)),
        out_shape=jax.ShapeDtypeStruct((Q, 1), jnp.float32),
    )(cand)


def _nks_summary(inp, emb_pad, lev_pad, lcen_pad, lbev, lbcen):
    t8, flag = pl.pallas_call(
        _stream_select_body,
        grid=(NB,),
        in_specs=[
            pl.BlockSpec((Q, D), lambda i: (0, 0)),
            pl.BlockSpec((EB, D), lambda i: (i, 0)),
        ],
        out_specs=[
            pl.BlockSpec((Q, 1), lambda i: (0, 0)),
            pl.BlockSpec((1, 1), lambda i: (0, 0)),
        ],
        out_shape=[
            jax.ShapeDtypeStruct((Q, 1), jnp.float32),
            jax.ShapeDtypeStruct((1, 1), jnp.float32),
        ],
        scratch_shapes=[pltpu.VMEM((Q, RDEPTH * NCLS), jnp.float32)],
    )(inp, emb_pad)
    t = lax.cond(flag[0, 0] > 0.0,
                 lambda: _deep_threshold(inp, emb_pad),
                 lambda: t8)

    out = pl.pallas_call(
        _accum_body,
        grid=(NB,),
        in_specs=[
            pl.BlockSpec((Q, D), lambda i: (0, 0)),
            pl.BlockSpec((EB, D), lambda i: (i, 0)),
            pl.BlockSpec((EB, T), lambda i: (i, 0)),
            pl.BlockSpec((EB, T), lambda i: (i, 0)),
            pl.BlockSpec((Q, 1), lambda i: (0, 0)),
            pl.BlockSpec((1, T), lambda i: (0, 0)),
            pl.BlockSpec((1, T), lambda i: (0, 0)),
        ],
        out_specs=pl.BlockSpec((Q, T), lambda i: (0, 0)),
        out_shape=jax.ShapeDtypeStruct((Q, T), jnp.float32),
        scratch_shapes=[
            pltpu.VMEM((Q, 2 * T), jnp.float32),
        ],
    )(inp, emb_pad, lev_pad, lcen_pad, t, lbev, lbcen)
    return out


def kernel(input, exemplar_embeddings, log_exemplar_event_counts,
           log_exemplar_censor_counts, log_baseline_event_counts,
           log_baseline_censor_counts):
    lbev = log_baseline_event_counts[None, :]
    lbcen = log_baseline_censor_counts[None, :]
    return _nks_summary(input, exemplar_embeddings,
                        log_exemplar_event_counts,
                        log_exemplar_censor_counts, lbev, lbcen)


# speedup vs baseline: 1.0872x; 1.0872x over previous
import functools

import jax
import jax.numpy as jnp
from jax import lax
from jax.experimental import pallas as pl
from jax.experimental.pallas import tpu as pltpu

KNN = 32
TAU2 = 4.0
Q = 1024
D = 32
T = 32
EB = 2048
E_RAW = 100000
NB = (E_RAW + EB - 1) // EB
E_PAD = NB * EB
BIG = 3.0e33

_DOT = functools.partial(
    lax.dot_general,
    preferred_element_type=jnp.float32,
    precision=lax.Precision.HIGHEST,
)


def _dist_block(q, e_raw, base):
    valid = (lax.broadcasted_iota(jnp.int32, (EB, 1), 0) + base) < E_RAW
    e = jnp.where(valid, e_raw, 0.0)
    q2 = jnp.sum(q * q, axis=1, keepdims=True)
    e2 = jnp.sum(e * e, axis=1)[None, :]
    e2 = jnp.where(valid.reshape(1, EB), e2, BIG)
    qe = lax.dot_general(q, e, (((1,), (1,)), ((), ())),
                         preferred_element_type=jnp.float32)
    return jnp.maximum(q2 + e2 - 2.0 * qe, 0.0)


DEPTH = 8
NCLS = 512
RDEPTH = 5


def _make_select(depth):
    def body(q_ref, e_ref, cand_ref):
        d2 = _dist_block(q_ref[...], e_ref[...], pl.program_id(0) * EB)
        cols = []
        for i in range(depth):
            m = jnp.min(d2, axis=1, keepdims=True)
            cols.append(m)
            if i + 1 < depth:
                d2 = jnp.where(d2 <= m, BIG, d2)
        cand_ref[...] = jnp.concatenate(cols, axis=1).reshape(1, Q, depth)
    return body


def _merge_body(cand_ref, t_ref):
    c = cand_ref[...]
    for _ in range(KNN - 1):
        m = jnp.min(c, axis=1, keepdims=True)
        c = jnp.where(c <= m, BIG, c)
    t = jnp.min(c, axis=1, keepdims=True)
    t_ref[...] = jnp.minimum(t, TAU2)


def _stream_select_body(q_ref, e_ref, r_ref):
    i = pl.program_id(0)

    @pl.when(i == 0)
    def _init():
        r_ref[...] = jnp.full_like(r_ref, BIG)

    d2 = _dist_block(q_ref[...], e_ref[...], i * EB)
    R = [r_ref[:, j * NCLS:(j + 1) * NCLS] for j in range(RDEPTH)]
    for c in range(EB // NCLS):
        x = d2[:, c * NCLS:(c + 1) * NCLS]
        for j in range(RDEPTH):
            lo = jnp.minimum(R[j], x)
            x = jnp.maximum(R[j], x)
            R[j] = lo
    for j in range(RDEPTH):
        r_ref[:, j * NCLS:(j + 1) * NCLS] = R[j]


def _merge_stream_body(cand_ref, t_ref, flag_ref):
    c = cand_ref[...]
    cm = c
    for _ in range(KNN - 1):
        m = jnp.min(cm, axis=1, keepdims=True)
        cm = jnp.where(cm <= m, BIG, cm)
    t = jnp.minimum(jnp.min(cm, axis=1, keepdims=True), TAU2)
    lane = lax.broadcasted_iota(jnp.int32, c.shape, 1)
    deepest = jnp.where(lane >= (RDEPTH - 1) * NCLS, c, BIG)
    tmin = jnp.min(deepest, axis=1, keepdims=True)
    t_ref[...] = t
    flag_ref[...] = jnp.max(jnp.where(tmin < t, 1.0, 0.0), axis=(0, 1),
                            keepdims=True)


def _accum_body(q_ref, e_ref, lev_ref, lcen_ref, t_ref, bev_ref, bcen_ref,
                out_ref, acc):
    i = pl.program_id(0)

    @pl.when(i == 0)
    def _init():
        acc[...] = jnp.zeros_like(acc)

    d2 = _dist_block(q_ref[...], e_ref[...], i * EB)
    w = jnp.where(d2 <= t_ref[...], jnp.exp(-d2), 0.0)
    valid = (lax.broadcasted_iota(jnp.int32, (EB, 1), 0) + i * EB) < E_RAW
    ev = jnp.where(valid, jnp.exp(lev_ref[...]), 0.0)
    tot = ev + jnp.where(valid, jnp.exp(lcen_ref[...]), 0.0)
    evtot = jnp.concatenate([ev, tot], axis=1)
    acc[...] += lax.dot_general(w, evtot, (((1,), (0,)), ((), ())),
                                preferred_element_type=jnp.float32)

    @pl.when(i == NB - 1)
    def _finish():
        ia = lax.broadcasted_iota(jnp.int32, (T, T), 0)
        ib = lax.broadcasted_iota(jnp.int32, (T, T), 1)
        M = (ia >= ib).astype(jnp.float32)
        bev = jnp.exp(bev_ref[...])
        btot = bev + jnp.exp(bcen_ref[...])
        brisk = _DOT(btot, M, (((1,), (0,)), ((), ())))
        numer = acc[:, :T] + bev
        denom = _DOT(acc[:, T:], M, (((1,), (0,)), ((), ()))) + brisk + 1e-12
        out_ref[...] = jnp.clip(numer / denom, 1e-12, 1.0 - 1e-12)


def _select_threshold(inp, emb_pad, depth):
    cand = pl.pallas_call(
        _make_select(depth),
        grid=(NB,),
        in_specs=[
            pl.BlockSpec((Q, D), lambda i: (0, 0)),
            pl.BlockSpec((EB, D), lambda i: (i, 0)),
        ],
        out_specs=pl.BlockSpec((1, Q, depth), lambda i: (i, 0, 0)),
        out_shape=jax.ShapeDtypeStruct((NB, Q, depth), jnp.float32),
    )(inp, emb_pad)
    return cand.transpose(1, 0, 2).reshape(Q, NB * depth)


def _deep_threshold(inp, emb_pad):
    cand = _select_threshold(inp, emb_pad, KNN)
    return pl.pallas_call(
        _merge_body,
        in_specs=[pl.BlockSpec((Q, NB * KNN), lambda: (0, 0))],
        out_specs=pl.BlockSpec((Q, 1), lambda: (0, 0)),
        out_shape=jax.ShapeDtypeStruct((Q, 1), jnp.float32),
    )(cand)


def _nks_summary(inp, emb_pad, lev_pad, lcen_pad, lbev, lbcen):
    cand = pl.pallas_call(
        _stream_select_body,
        grid=(NB,),
        in_specs=[
            pl.BlockSpec((Q, D), lambda i: (0, 0)),
            pl.BlockSpec((EB, D), lambda i: (i, 0)),
        ],
        out_specs=pl.BlockSpec((Q, RDEPTH * NCLS), lambda i: (0, 0)),
        out_shape=jax.ShapeDtypeStruct((Q, RDEPTH * NCLS), jnp.float32),
    )(inp, emb_pad)
    t8, flag = pl.pallas_call(
        _merge_stream_body,
        in_specs=[pl.BlockSpec((Q, RDEPTH * NCLS), lambda: (0, 0))],
        out_specs=[
            pl.BlockSpec((Q, 1), lambda: (0, 0)),
            pl.BlockSpec((1, 1), lambda: (0, 0)),
        ],
        out_shape=[
            jax.ShapeDtypeStruct((Q, 1), jnp.float32),
            jax.ShapeDtypeStruct((1, 1), jnp.float32),
        ],
    )(cand)
    t = lax.cond(flag[0, 0] > 0.0,
                 lambda: _deep_threshold(inp, emb_pad),
                 lambda: t8)

    out = pl.pallas_call(
        _accum_body,
        grid=(NB,),
        in_specs=[
            pl.BlockSpec((Q, D), lambda i: (0, 0)),
            pl.BlockSpec((EB, D), lambda i: (i, 0)),
            pl.BlockSpec((EB, T), lambda i: (i, 0)),
            pl.BlockSpec((EB, T), lambda i: (i, 0)),
            pl.BlockSpec((Q, 1), lambda i: (0, 0)),
            pl.BlockSpec((1, T), lambda i: (0, 0)),
            pl.BlockSpec((1, T), lambda i: (0, 0)),
        ],
        out_specs=pl.BlockSpec((Q, T), lambda i: (0, 0)),
        out_shape=jax.ShapeDtypeStruct((Q, T), jnp.float32),
        scratch_shapes=[
            pltpu.VMEM((Q, 2 * T), jnp.float32),
        ],
    )(inp, emb_pad, lev_pad, lcen_pad, t, lbev, lbcen)
    return out


def kernel(input, exemplar_embeddings, log_exemplar_event_counts,
           log_exemplar_censor_counts, log_baseline_event_counts,
           log_baseline_censor_counts):
    lbev = log_baseline_event_counts[None, :]
    lbcen = log_baseline_censor_counts[None, :]
    return _nks_summary(input, exemplar_embeddings,
                        log_exemplar_event_counts,
                        log_exemplar_censor_counts, lbev, lbcen)


# speedup vs baseline: 1.0921x; 1.0045x over previous
import functools

import jax
import jax.numpy as jnp
from jax import lax
from jax.experimental import pallas as pl
from jax.experimental.pallas import tpu as pltpu

KNN = 32
TAU2 = 4.0
Q = 1024
D = 32
T = 32
EB = 2048
E_RAW = 100000
NB = (E_RAW + EB - 1) // EB
E_PAD = NB * EB
BIG = 3.0e33

_DOT = functools.partial(
    lax.dot_general,
    preferred_element_type=jnp.float32,
    precision=lax.Precision.HIGHEST,
)


def _dist_block(q, e_raw, base):
    valid = (lax.broadcasted_iota(jnp.int32, (EB, 1), 0) + base) < E_RAW
    e = jnp.where(valid, e_raw, 0.0)
    q2 = jnp.sum(q * q, axis=1, keepdims=True)
    e2 = jnp.sum(e * e, axis=1)[None, :]
    e2 = jnp.where(valid.reshape(1, EB), e2, BIG)
    qe = lax.dot_general(q, e, (((1,), (1,)), ((), ())),
                         preferred_element_type=jnp.float32)
    return jnp.maximum(q2 + e2 - 2.0 * qe, 0.0)


DEPTH = 8
NCLS = 1024
RDEPTH = 4


def _make_select(depth):
    def body(q_ref, e_ref, cand_ref):
        d2 = _dist_block(q_ref[...], e_ref[...], pl.program_id(0) * EB)
        cols = []
        for i in range(depth):
            m = jnp.min(d2, axis=1, keepdims=True)
            cols.append(m)
            if i + 1 < depth:
                d2 = jnp.where(d2 <= m, BIG, d2)
        cand_ref[...] = jnp.concatenate(cols, axis=1).reshape(1, Q, depth)
    return body


def _merge_body(cand_ref, t_ref):
    c = cand_ref[...]
    for _ in range(KNN - 1):
        m = jnp.min(c, axis=1, keepdims=True)
        c = jnp.where(c <= m, BIG, c)
    t = jnp.min(c, axis=1, keepdims=True)
    t_ref[...] = jnp.minimum(t, TAU2)


def _stream_select_body(q_ref, e_ref, r_ref):
    i = pl.program_id(0)

    @pl.when(i == 0)
    def _init():
        r_ref[...] = jnp.full_like(r_ref, BIG)

    d2 = _dist_block(q_ref[...], e_ref[...], i * EB)
    R = [r_ref[:, j * NCLS:(j + 1) * NCLS] for j in range(RDEPTH)]
    for c in range(EB // NCLS):
        x = d2[:, c * NCLS:(c + 1) * NCLS]
        for j in range(RDEPTH):
            lo = jnp.minimum(R[j], x)
            x = jnp.maximum(R[j], x)
            R[j] = lo
    for j in range(RDEPTH):
        r_ref[:, j * NCLS:(j + 1) * NCLS] = R[j]


def _merge_stream_body(cand_ref, t_ref, flag_ref):
    c = cand_ref[...]
    cm = c
    for _ in range(KNN - 1):
        m = jnp.min(cm, axis=1, keepdims=True)
        cm = jnp.where(cm <= m, BIG, cm)
    t = jnp.minimum(jnp.min(cm, axis=1, keepdims=True), TAU2)
    lane = lax.broadcasted_iota(jnp.int32, c.shape, 1)
    deepest = jnp.where(lane >= (RDEPTH - 1) * NCLS, c, BIG)
    tmin = jnp.min(deepest, axis=1, keepdims=True)
    t_ref[...] = t
    flag_ref[...] = jnp.max(jnp.where(tmin < t, 1.0, 0.0), axis=(0, 1),
                            keepdims=True)


def _accum_body(q_ref, e_ref, lev_ref, lcen_ref, t_ref, bev_ref, bcen_ref,
                out_ref, acc):
    i = pl.program_id(0)

    @pl.when(i == 0)
    def _init():
        acc[...] = jnp.zeros_like(acc)

    d2 = _dist_block(q_ref[...], e_ref[...], i * EB)
    w = jnp.where(d2 <= t_ref[...], jnp.exp(-d2), 0.0)
    valid = (lax.broadcasted_iota(jnp.int32, (EB, 1), 0) + i * EB) < E_RAW
    ev = jnp.where(valid, jnp.exp(lev_ref[...]), 0.0)
    tot = ev + jnp.where(valid, jnp.exp(lcen_ref[...]), 0.0)
    evtot = jnp.concatenate([ev, tot], axis=1)
    acc[...] += lax.dot_general(w, evtot, (((1,), (0,)), ((), ())),
                                preferred_element_type=jnp.float32)

    @pl.when(i == NB - 1)
    def _finish():
        ia = lax.broadcasted_iota(jnp.int32, (T, T), 0)
        ib = lax.broadcasted_iota(jnp.int32, (T, T), 1)
        M = (ia >= ib).astype(jnp.float32)
        bev = jnp.exp(bev_ref[...])
        btot = bev + jnp.exp(bcen_ref[...])
        brisk = _DOT(btot, M, (((1,), (0,)), ((), ())))
        numer = acc[:, :T] + bev
        denom = _DOT(acc[:, T:], M, (((1,), (0,)), ((), ()))) + brisk + 1e-12
        out_ref[...] = jnp.clip(numer / denom, 1e-12, 1.0 - 1e-12)


def _select_threshold(inp, emb_pad, depth):
    cand = pl.pallas_call(
        _make_select(depth),
        grid=(NB,),
        in_specs=[
            pl.BlockSpec((Q, D), lambda i: (0, 0)),
            pl.BlockSpec((EB, D), lambda i: (i, 0)),
        ],
        out_specs=pl.BlockSpec((1, Q, depth), lambda i: (i, 0, 0)),
        out_shape=jax.ShapeDtypeStruct((NB, Q, depth), jnp.float32),
    )(inp, emb_pad)
    return cand.transpose(1, 0, 2).reshape(Q, NB * depth)


def _deep_threshold(inp, emb_pad):
    cand = _select_threshold(inp, emb_pad, KNN)
    return pl.pallas_call(
        _merge_body,
        in_specs=[pl.BlockSpec((Q, NB * KNN), lambda: (0, 0))],
        out_specs=pl.BlockSpec((Q, 1), lambda: (0, 0)),
        out_shape=jax.ShapeDtypeStruct((Q, 1), jnp.float32),
    )(cand)


def _nks_summary(inp, emb_pad, lev_pad, lcen_pad, lbev, lbcen):
    cand = pl.pallas_call(
        _stream_select_body,
        grid=(NB,),
        in_specs=[
            pl.BlockSpec((Q, D), lambda i: (0, 0)),
            pl.BlockSpec((EB, D), lambda i: (i, 0)),
        ],
        out_specs=pl.BlockSpec((Q, RDEPTH * NCLS), lambda i: (0, 0)),
        out_shape=jax.ShapeDtypeStruct((Q, RDEPTH * NCLS), jnp.float32),
    )(inp, emb_pad)
    t8, flag = pl.pallas_call(
        _merge_stream_body,
        in_specs=[pl.BlockSpec((Q, RDEPTH * NCLS), lambda: (0, 0))],
        out_specs=[
            pl.BlockSpec((Q, 1), lambda: (0, 0)),
            pl.BlockSpec((1, 1), lambda: (0, 0)),
        ],
        out_shape=[
            jax.ShapeDtypeStruct((Q, 1), jnp.float32),
            jax.ShapeDtypeStruct((1, 1), jnp.float32),
        ],
    )(cand)
    t = lax.cond(flag[0, 0] > 0.0,
                 lambda: _deep_threshold(inp, emb_pad),
                 lambda: t8)

    out = pl.pallas_call(
        _accum_body,
        grid=(NB,),
        in_specs=[
            pl.BlockSpec((Q, D), lambda i: (0, 0)),
            pl.BlockSpec((EB, D), lambda i: (i, 0)),
            pl.BlockSpec((EB, T), lambda i: (i, 0)),
            pl.BlockSpec((EB, T), lambda i: (i, 0)),
            pl.BlockSpec((Q, 1), lambda i: (0, 0)),
            pl.BlockSpec((1, T), lambda i: (0, 0)),
            pl.BlockSpec((1, T), lambda i: (0, 0)),
        ],
        out_specs=pl.BlockSpec((Q, T), lambda i: (0, 0)),
        out_shape=jax.ShapeDtypeStruct((Q, T), jnp.float32),
        scratch_shapes=[
            pltpu.VMEM((Q, 2 * T), jnp.float32),
        ],
    )(inp, emb_pad, lev_pad, lcen_pad, t, lbev, lbcen)
    return out


def kernel(input, exemplar_embeddings, log_exemplar_event_counts,
           log_exemplar_censor_counts, log_baseline_event_counts,
           log_baseline_censor_counts):
    lbev = log_baseline_event_counts[None, :]
    lbcen = log_baseline_censor_counts[None, :]
    return _nks_summary(input, exemplar_embeddings,
                        log_exemplar_event_counts,
                        log_exemplar_censor_counts, lbev, lbcen)
